# Initial kernel scaffold; baseline (speedup 1.0000x reference)
import jax
import jax.numpy as jnp
from jax.experimental import pallas as pl


def kernel(pieces, tree_pos_path, tree_neg_path, emb_table, W, b):
    raise NotImplementedError("write your pallas kernel here")


# speedup vs baseline: 2.1583x; 2.1583x over previous
import functools

import jax
import jax.numpy as jnp
from jax import lax
from jax.experimental import pallas as pl
from jax.experimental.pallas import tpu as pltpu
from jax.experimental.pallas import tpu_sc as plsc

VOCAB = 1000000
D = 32
B = 4096
S = 200
T = 1024
NC, NS = 2, 16
NW = NC * NS
BW = B // NW
HALF = S // 2
NBUF = 4


def _sc_gather_sum(table, pieces3):
    mesh = plsc.VectorSubcoreMesh(
        core_axis_name="c", subcore_axis_name="s", num_cores=NC, num_subcores=NS
    )

    @functools.partial(
        pl.kernel,
        out_type=jax.ShapeDtypeStruct((B, D), jnp.float32),
        mesh=mesh,
        scratch_types=[
            pltpu.VMEM((BW, 2, HALF), jnp.int32),
            pltpu.VMEM((NBUF, S, D), jnp.float32),
            pltpu.VMEM((BW, D), jnp.float32),
            pltpu.SemaphoreType.DMA,
            pltpu.SemaphoreType.DMA,
            pltpu.SemaphoreType.DMA,
            pltpu.SemaphoreType.DMA,
        ],
        compiler_params=pltpu.CompilerParams(use_tc_tiling_on_sc=False),
    )
    def k(table_hbm, idx_hbm, out_hbm, idx_v, rows_v, feat_v, sem0, sem1, sem2, sem3):
        sems = (sem0, sem1, sem2, sem3)
        wid = lax.axis_index("s") * NC + lax.axis_index("c")
        base = wid * BW
        pltpu.sync_copy(idx_hbm.at[pl.ds(base, BW)], idx_v)

        def issue(r, buf):
            for c2 in range(2):
                pltpu.async_copy(
                    table_hbm.at[idx_v.at[r, c2]],
                    rows_v.at[buf, pl.ds(c2 * HALF, HALF)],
                    sems[buf],
                )

        def wait(buf):
            for c2 in range(2):
                pltpu.make_async_copy(
                    table_hbm.at[idx_v.at[0, c2]],
                    rows_v.at[buf, pl.ds(c2 * HALF, HALF)],
                    sems[buf],
                ).wait()

        def accumulate(r, buf):
            rows = rows_v.at[buf]
            zero = jnp.zeros((16,), jnp.float32)
            acc_a = [zero, zero, zero, zero]
            acc_b = [zero, zero, zero, zero]
            for j in range(S):
                k4 = j % 4
                acc_a[k4] = acc_a[k4] + rows[j, pl.ds(0, 16)]
                acc_b[k4] = acc_b[k4] + rows[j, pl.ds(16, 16)]
            feat_v[r, pl.ds(0, 16)] = (acc_a[0] + acc_a[1]) + (acc_a[2] + acc_a[3])
            feat_v[r, pl.ds(16, 16)] = (acc_b[0] + acc_b[1]) + (acc_b[2] + acc_b[3])

        for buf in range(NBUF):
            issue(buf, buf)

        @pl.loop(0, BW - NBUF, step=NBUF)
        def _(r0):
            for buf in range(NBUF):
                r = r0 + buf
                wait(buf)
                accumulate(r, buf)
                issue(r + NBUF, buf)

        for buf in range(NBUF):
            wait(buf)
            accumulate(BW - NBUF + buf, buf)

        pltpu.sync_copy(feat_v, out_hbm.at[pl.ds(base, BW)])

    return k(table, pieces3)


def _tc_tail(feature, pieces, pos, neg, w, b2, t0):
    BB = 256

    def body(feat_ref, pieces_ref, pos_ref, neg_ref, w_ref, b_ref, t0_ref, out_ref):
        cnt0 = jnp.sum((pieces_ref[...] == 0).astype(jnp.float32), axis=1)
        feat = feat_ref[...] - cnt0[:, None] * t0_ref[...]
        logits = lax.dot_general(
            feat, w_ref[...], (((1,), (1,)), ((), ())),
            preferred_element_type=jnp.float32,
        ) + b_ref[...]
        trans = 1.0 / (1.0 + jnp.exp(-logits))
        p = pos_ref[...]
        n = neg_ref[...]
        out_ref[...] = jnp.sum((p - n) * trans + n, axis=1)

    return pl.pallas_call(
        body,
        grid=(B // BB,),
        in_specs=[
            pl.BlockSpec((BB, D), lambda i: (i, 0)),
            pl.BlockSpec((BB, S), lambda i: (i, 0)),
            pl.BlockSpec((BB, T), lambda i: (i, 0)),
            pl.BlockSpec((BB, T), lambda i: (i, 0)),
            pl.BlockSpec((T, D), lambda i: (0, 0)),
            pl.BlockSpec((1, T), lambda i: (0, 0)),
            pl.BlockSpec((1, D), lambda i: (0, 0)),
        ],
        out_specs=pl.BlockSpec((BB,), lambda i: (i,)),
        out_shape=jax.ShapeDtypeStruct((B,), jnp.float32),
    )(feature, pieces, pos, neg, w, b2, t0)


def kernel(pieces, tree_pos_path, tree_neg_path, emb_table, W, b):
    pieces = pieces.astype(jnp.int32)
    pieces3 = pieces.reshape(B, 2, HALF)
    feature = _sc_gather_sum(emb_table, pieces3)
    b2 = b.reshape(1, T)
    t0 = lax.slice(emb_table, (0, 0), (1, D))
    return _tc_tail(feature, pieces, tree_pos_path, tree_neg_path, W, b2, t0)
